# baseline (speedup 1.0000x reference)
import functools

import jax
import jax.numpy as jnp
from jax import lax
from jax.experimental import pallas as pl
from jax.experimental.pallas import tpu as pltpu
from jax.experimental.pallas import tpu_sc as plsc

N = 10000
NP = 10112
E = 320000
NC = 2
NS = 16
NW = NC * NS
CH = 128
NCH = 80
NBUF = 4
EPAD = NW * NCH * CH
RPS = NP // NS

_f32 = jnp.float32
_i32 = jnp.int32


def _sc_mesh():
    return plsc.VectorSubcoreMesh(core_axis_name="c", subcore_axis_name="s")


@functools.partial(
    pl.kernel,
    out_type=jax.ShapeDtypeStruct((NC, NP, 16), _f32),
    mesh=_sc_mesh(),
    compiler_params=pltpu.CompilerParams(use_tc_tiling_on_sc=False),
    scratch_types=[
        pltpu.VMEM((NCH, CH), _i32),
        pltpu.VMEM((CH, 16), _f32),
        pltpu.VMEM_SHARED((NP, 16), _f32),
    ],
)
def _sc_deg(dst_hbm, zeros_hbm, ones_hbm, out_hbm, dst_v, ones_v, acc):
    c = lax.axis_index("c")
    s = lax.axis_index("s")
    wid = c * NS + s
    pltpu.sync_copy(zeros_hbm.at[pl.ds(s * RPS, RPS)], acc.at[pl.ds(s * RPS, RPS)])
    pltpu.sync_copy(dst_hbm.at[wid], dst_v)
    pltpu.sync_copy(ones_hbm, ones_v)
    plsc.subcore_barrier()

    def body(j, carry):
        pltpu.sync_copy(ones_v, acc.at[dst_v.at[j]], add=True)
        return carry

    lax.fori_loop(0, NCH, body, 0)
    plsc.subcore_barrier()
    pltpu.sync_copy(acc.at[pl.ds(s * RPS, RPS)], out_hbm.at[c, pl.ds(s * RPS, RPS)])


def _make_sc_agg(F, dtype, stage=False, nbuf=NBUF):
    scratch = [
        pltpu.VMEM((NCH, CH), _i32),
        pltpu.VMEM((NCH, CH), _i32),
        pltpu.VMEM((nbuf, CH, F), dtype),
        pltpu.VMEM_SHARED((NP, F), dtype),
    ]
    if stage:
        scratch.append(pltpu.VMEM_SHARED((NP, F), dtype))
    scratch += [pltpu.SemaphoreType.DMA] * nbuf

    @functools.partial(
        pl.kernel,
        out_type=jax.ShapeDtypeStruct((NC, NP, F), dtype),
        mesh=_sc_mesh(),
        compiler_params=pltpu.CompilerParams(use_tc_tiling_on_sc=False),
        scratch_types=scratch,
    )
    def agg(g_hbm, src_hbm, dst_hbm, zeros_hbm, out_hbm,
            src_v, dst_v, rows, acc, *rest):
        if stage:
            g_sp, gsems = rest[0], rest[1:]
        else:
            g_sp, gsems = None, rest
        c = lax.axis_index("c")
        s = lax.axis_index("s")
        wid = c * NS + s
        pltpu.sync_copy(zeros_hbm.at[pl.ds(s * RPS, RPS)], acc.at[pl.ds(s * RPS, RPS)])
        pltpu.sync_copy(src_hbm.at[wid], src_v)
        pltpu.sync_copy(dst_hbm.at[wid], dst_v)
        if stage:
            pltpu.sync_copy(g_hbm.at[pl.ds(s * RPS, RPS)],
                            g_sp.at[pl.ds(s * RPS, RPS)])
        plsc.subcore_barrier()
        gsrc = g_sp if stage else g_hbm

        def g_start(j, b):
            pltpu.async_copy(gsrc.at[src_v.at[j]], rows.at[b], gsems[b])

        def g_wait(j, b):
            pltpu.make_async_copy(gsrc.at[src_v.at[j]], rows.at[b], gsems[b]).wait()

        def scat(j, b):
            pltpu.sync_copy(rows.at[b], acc.at[dst_v.at[j]], add=True)

        for b in range(nbuf):
            g_start(b, b)

        def body(i, carry):
            for b in range(nbuf):
                j = nbuf * i + b
                g_wait(j, b)
                scat(j, b)
                g_start(j + nbuf, b)
            return carry

        lax.fori_loop(0, NCH // nbuf - 1, body, 0)
        for b in range(nbuf):
            j = NCH - nbuf + b
            g_wait(j, b)
            scat(j, b)

        plsc.subcore_barrier()
        pltpu.sync_copy(acc.at[pl.ds(s * RPS, RPS)], out_hbm.at[c, pl.ds(s * RPS, RPS)])

    return agg


_sc_agg128 = _make_sc_agg(128, jnp.bfloat16, stage=True, nbuf=2)
_sc_agg16 = _make_sc_agg(16, _f32)


def _tc_prep(xpad, W1, deg_parts):
    def body(x_ref, w_ref, dp_ref, g_ref, dis_ref):
        h = jnp.dot(x_ref[...], w_ref[...],
                    preferred_element_type=_f32,
                    precision=lax.Precision.HIGHEST)
        degf = dp_ref[0] + dp_ref[1] + 1.0
        disf = lax.rsqrt(degf)
        dis_ref[...] = disf
        g_ref[...] = (h * disf[:, 0:1]).astype(jnp.bfloat16)

    return pl.pallas_call(
        body,
        out_shape=(jax.ShapeDtypeStruct((NP, 128), jnp.bfloat16),
                   jax.ShapeDtypeStruct((NP, 16), _f32)),
    )(xpad, W1, deg_parts)


def _tc_mid(parts1, g1, disf, b1, W2):
    def body(p_ref, g_ref, d_ref, b_ref, w_ref, g2_ref):
        dis = d_ref[:, 0:1]
        agg = (p_ref[0].astype(_f32) + p_ref[1].astype(_f32)
               + g_ref[...].astype(_f32))
        out1 = agg * dis + b_ref[...][None, :]
        out1 = jnp.maximum(out1, 0.0)
        h2 = jnp.dot(out1, w_ref[...],
                     preferred_element_type=_f32,
                     precision=lax.Precision.HIGHEST)
        g2_ref[...] = h2 * dis

    return pl.pallas_call(
        body,
        out_shape=jax.ShapeDtypeStruct((NP, 16), _f32),
    )(parts1, g1, disf, b1, W2)


def _tc_final(parts2, g2, disf, b2):
    def body(p_ref, g_ref, d_ref, b_ref, o_ref):
        agg = (p_ref[0].astype(_f32) + p_ref[1].astype(_f32)
               + g_ref[...].astype(_f32))
        o_ref[...] = agg * d_ref[:, 0:1] + b_ref[...][None, :]

    return pl.pallas_call(
        body,
        out_shape=jax.ShapeDtypeStruct((NP, 16), _f32),
    )(parts2, g2, disf, b2)


@jax.jit
def kernel(features, edge_index, W1, b1, W2, b2):
    src = edge_index[0].astype(_i32)
    dst = edge_index[1].astype(_i32)
    pad = N + (jnp.arange(EPAD - E, dtype=_i32) % (NP - N))
    src3 = jnp.concatenate([src, pad]).reshape(NW, NCH, CH)
    dst3 = jnp.concatenate([dst, pad]).reshape(NW, NCH, CH)

    xpad = jnp.pad(features, ((0, NP - N), (0, 0)))
    zeros16 = jnp.zeros((NP, 16), _f32)
    zeros128 = jnp.zeros((NP, 128), jnp.bfloat16)
    ones16 = jnp.ones((CH, 16), _f32)

    deg_parts = _sc_deg(dst3, zeros16, ones16)
    g1, disf = _tc_prep(xpad, W1, deg_parts)
    parts1 = _sc_agg128(g1, src3, dst3, zeros128)
    g2 = _tc_mid(parts1, g1, disf, b1, W2)
    parts2 = _sc_agg16(g2, src3, dst3, zeros16)
    out = _tc_final(parts2, g2, disf, b2)
    return out[:N]


# speedup vs baseline: 1.1247x; 1.1247x over previous
import functools

import jax
import jax.numpy as jnp
from jax import lax
from jax.experimental import pallas as pl
from jax.experimental.pallas import tpu as pltpu
from jax.experimental.pallas import tpu_sc as plsc

N = 10000
NP = 10112
E = 320000
NC = 2
NS = 16
NW = NC * NS
CH = 128
NCH = 80
NBUF = 4
EPAD = NW * NCH * CH
RPS = NP // NS

_f32 = jnp.float32
_i32 = jnp.int32


def _sc_mesh():
    return plsc.VectorSubcoreMesh(core_axis_name="c", subcore_axis_name="s")


@functools.partial(
    pl.kernel,
    out_type=jax.ShapeDtypeStruct((NC, NP, 16), _f32),
    mesh=_sc_mesh(),
    compiler_params=pltpu.CompilerParams(use_tc_tiling_on_sc=False),
    scratch_types=[
        pltpu.VMEM((NCH, CH), _i32),
        pltpu.VMEM((CH, 16), _f32),
        pltpu.VMEM_SHARED((NP, 16), _f32),
    ],
)
def _sc_deg(dst_hbm, zeros_hbm, ones_hbm, out_hbm, dst_v, ones_v, acc):
    c = lax.axis_index("c")
    s = lax.axis_index("s")
    wid = c * NS + s
    pltpu.sync_copy(zeros_hbm.at[pl.ds(s * RPS, RPS)], acc.at[pl.ds(s * RPS, RPS)])
    pltpu.sync_copy(dst_hbm.at[wid], dst_v)
    pltpu.sync_copy(ones_hbm, ones_v)
    plsc.subcore_barrier()

    def body(j, carry):
        pltpu.sync_copy(ones_v, acc.at[dst_v.at[j]], add=True)
        return carry

    lax.fori_loop(0, NCH, body, 0)
    plsc.subcore_barrier()
    pltpu.sync_copy(acc.at[pl.ds(s * RPS, RPS)], out_hbm.at[c, pl.ds(s * RPS, RPS)])


def _make_sc_agg(F, dtype, stage=False, nbuf=NBUF):
    scratch = [
        pltpu.VMEM((NCH, CH), _i32),
        pltpu.VMEM((NCH, CH), _i32),
        pltpu.VMEM((nbuf, CH, F), dtype),
        pltpu.VMEM_SHARED((NP, F), dtype),
    ]
    if stage:
        scratch.append(pltpu.VMEM_SHARED((NP, F), dtype))
    scratch += [pltpu.SemaphoreType.DMA] * nbuf

    @functools.partial(
        pl.kernel,
        out_type=jax.ShapeDtypeStruct((NC, NP, F), dtype),
        mesh=_sc_mesh(),
        compiler_params=pltpu.CompilerParams(use_tc_tiling_on_sc=False),
        scratch_types=scratch,
    )
    def agg(g_hbm, src_hbm, dst_hbm, zeros_hbm, out_hbm,
            src_v, dst_v, rows, acc, *rest):
        if stage:
            g_sp, gsems = rest[0], rest[1:]
        else:
            g_sp, gsems = None, rest
        c = lax.axis_index("c")
        s = lax.axis_index("s")
        wid = c * NS + s
        pltpu.sync_copy(zeros_hbm.at[pl.ds(s * RPS, RPS)], acc.at[pl.ds(s * RPS, RPS)])
        pltpu.sync_copy(src_hbm.at[wid], src_v)
        pltpu.sync_copy(dst_hbm.at[wid], dst_v)
        if stage:
            pltpu.sync_copy(g_hbm.at[pl.ds(s * RPS, RPS)],
                            g_sp.at[pl.ds(s * RPS, RPS)])
        plsc.subcore_barrier()
        gsrc = g_sp if stage else g_hbm

        def g_start(j, b):
            pltpu.async_copy(gsrc.at[src_v.at[j]], rows.at[b], gsems[b])

        def g_wait(j, b):
            pltpu.make_async_copy(gsrc.at[src_v.at[j]], rows.at[b], gsems[b]).wait()

        def scat(j, b):
            pltpu.sync_copy(rows.at[b], acc.at[dst_v.at[j]], add=True)

        for b in range(nbuf):
            g_start(b, b)

        def body(i, carry):
            for b in range(nbuf):
                j = nbuf * i + b
                g_wait(j, b)
                scat(j, b)
                g_start(j + nbuf, b)
            return carry

        lax.fori_loop(0, NCH // nbuf - 1, body, 0)
        for b in range(nbuf):
            j = NCH - nbuf + b
            g_wait(j, b)
            scat(j, b)

        plsc.subcore_barrier()
        pltpu.sync_copy(acc.at[pl.ds(s * RPS, RPS)], out_hbm.at[c, pl.ds(s * RPS, RPS)])

    return agg


_sc_agg128 = _make_sc_agg(128, jnp.bfloat16)
_sc_agg16 = _make_sc_agg(16, _f32)


def _tc_matmul(xpad, W1):
    def body(x_ref, w_ref, h_ref):
        h_ref[...] = jnp.dot(x_ref[...], w_ref[...],
                             preferred_element_type=_f32,
                             precision=lax.Precision.HIGHEST)

    return pl.pallas_call(
        body,
        out_shape=jax.ShapeDtypeStruct((NP, 128), _f32),
    )(xpad, W1)


def _tc_prep(h1, deg_parts):
    def body(h_ref, dp_ref, g_ref, dis_ref):
        degf = dp_ref[0] + dp_ref[1] + 1.0
        disf = lax.rsqrt(degf)
        dis_ref[...] = disf
        g_ref[...] = (h_ref[...] * disf[:, 0:1]).astype(jnp.bfloat16)

    return pl.pallas_call(
        body,
        out_shape=(jax.ShapeDtypeStruct((NP, 128), jnp.bfloat16),
                   jax.ShapeDtypeStruct((NP, 16), _f32)),
    )(h1, deg_parts)


def _tc_mid(parts1, g1, disf, b1, W2):
    def body(p_ref, g_ref, d_ref, b_ref, w_ref, g2_ref):
        dis = d_ref[:, 0:1]
        agg = (p_ref[0].astype(_f32) + p_ref[1].astype(_f32)
               + g_ref[...].astype(_f32))
        out1 = agg * dis + b_ref[...][None, :]
        out1 = jnp.maximum(out1, 0.0)
        h2 = jnp.dot(out1, w_ref[...],
                     preferred_element_type=_f32,
                     precision=lax.Precision.HIGHEST)
        g2_ref[...] = h2 * dis

    return pl.pallas_call(
        body,
        out_shape=jax.ShapeDtypeStruct((NP, 16), _f32),
    )(parts1, g1, disf, b1, W2)


def _tc_final(parts2, g2, disf, b2):
    def body(p_ref, g_ref, d_ref, b_ref, o_ref):
        agg = (p_ref[0].astype(_f32) + p_ref[1].astype(_f32)
               + g_ref[...].astype(_f32))
        o_ref[...] = agg * d_ref[:, 0:1] + b_ref[...][None, :]

    return pl.pallas_call(
        body,
        out_shape=jax.ShapeDtypeStruct((NP, 16), _f32),
    )(parts2, g2, disf, b2)


@jax.jit
def kernel(features, edge_index, W1, b1, W2, b2):
    src = edge_index[0].astype(_i32)
    dst = edge_index[1].astype(_i32)
    pad = N + (jnp.arange(EPAD - E, dtype=_i32) % (NP - N))
    src3 = jnp.concatenate([src, pad]).reshape(NW, NCH, CH)
    dst3 = jnp.concatenate([dst, pad]).reshape(NW, NCH, CH)

    xpad = jnp.pad(features, ((0, NP - N), (0, 0)))
    zeros16 = jnp.zeros((NP, 16), _f32)
    zeros128 = jnp.zeros((NP, 128), jnp.bfloat16)
    ones16 = jnp.ones((CH, 16), _f32)

    h1 = _tc_matmul(xpad, W1)
    deg_parts = _sc_deg(dst3, zeros16, ones16)
    g1, disf = _tc_prep(h1, deg_parts)
    parts1 = _sc_agg128(g1, src3, dst3, zeros128)
    g2 = _tc_mid(parts1, g1, disf, b1, W2)
    parts2 = _sc_agg16(g2, src3, dst3, zeros16)
    out = _tc_final(parts2, g2, disf, b2)
    return out[:N]
